# baseline (speedup 1.0000x reference)
import jax
import jax.numpy as jnp
import numpy as np
from jax.experimental import pallas as pl
from jax.experimental.pallas import tpu as pltpu

K = 2
ENC = 1024
IN_DIM = 2048
OUT_DIM = 4096
HID = 512
BLK_M = 128
SEQ_BLK = K * BLK_M

_SEL = np.zeros((K, BLK_M, SEQ_BLK), dtype=np.float32)
for _r in range(BLK_M):
    _SEL[0, _r, K * _r] = 1.0
    _SEL[1, _r, K * _r + 1] = 1.0


def _fused_kernel(x_ref, se_ref, so_ref, g1_ref, b1_ref, w12_ref, w3_ref,
                  g2_ref, b2_ref, out_ref, w12b_ref, w3b_ref):
    @pl.when(jnp.logical_and(pl.program_id(0) == 0, pl.program_id(1) == 0))
    def _cast_weights():
        w12b_ref[...] = w12_ref[...].astype(jnp.bfloat16)
        w3b_ref[...] = w3_ref[...].astype(jnp.bfloat16)

    xb = x_ref[0].astype(jnp.bfloat16)
    xe = jax.lax.dot_general(se_ref[...], xb, (((1,), (0,)), ((), ())),
                             preferred_element_type=jnp.float32)
    xo = jax.lax.dot_general(so_ref[...], xb, (((1,), (0,)), ((), ())),
                             preferred_element_type=jnp.float32)
    mean = (jnp.sum(xe, axis=-1, keepdims=True)
            + jnp.sum(xo, axis=-1, keepdims=True)) * (1.0 / IN_DIM)
    xce = xe - mean
    xco = xo - mean
    var = (jnp.sum(xce * xce, axis=-1, keepdims=True)
           + jnp.sum(xco * xco, axis=-1, keepdims=True)) * (1.0 / IN_DIM)
    rs = jax.lax.rsqrt(var + 1e-6)
    xne = (xce * rs * g1_ref[0:1, :] + b1_ref[0:1, :]).astype(jnp.bfloat16)
    xno = (xco * rs * g1_ref[1:2, :] + b1_ref[1:2, :]).astype(jnp.bfloat16)
    h = (jax.lax.dot_general(xne, w12b_ref[:, :ENC], (((1,), (1,)), ((), ())),
                             preferred_element_type=jnp.float32)
         + jax.lax.dot_general(xno, w12b_ref[:, ENC:],
                               (((1,), (1,)), ((), ())),
                               preferred_element_type=jnp.float32))
    gate = h[:, :HID]
    val = h[:, HID:]
    act = (gate * jax.nn.sigmoid(gate) * val).astype(jnp.bfloat16)
    y = jax.lax.dot_general(act, w3b_ref[...], (((1,), (1,)), ((), ())),
                            preferred_element_type=jnp.float32)
    mean2 = jnp.mean(y, axis=-1, keepdims=True)
    yc = y - mean2
    var2 = jnp.mean(yc * yc, axis=-1, keepdims=True)
    out_ref[0] = yc * jax.lax.rsqrt(var2 + 1e-6) * g2_ref[...] + b2_ref[...]


def kernel(x, ln_pre_g, ln_pre_b, w12, w3, router_w, router_b, ln_post_g,
           ln_post_b):
    b, s, d = x.shape
    m = s // K
    nb = m // BLK_M
    sel = jnp.asarray(_SEL, dtype=jnp.bfloat16)
    out = pl.pallas_call(
        _fused_kernel,
        grid=(b, nb),
        in_specs=[
            pl.BlockSpec((1, SEQ_BLK, ENC), lambda i, j: (i, j, 0)),
            pl.BlockSpec((BLK_M, SEQ_BLK), lambda i, j: (0, 0)),
            pl.BlockSpec((BLK_M, SEQ_BLK), lambda i, j: (0, 0)),
            pl.BlockSpec((K, IN_DIM // K), lambda i, j: (0, 0)),
            pl.BlockSpec((K, IN_DIM // K), lambda i, j: (0, 0)),
            pl.BlockSpec((2 * HID, IN_DIM), lambda i, j: (0, 0)),
            pl.BlockSpec((OUT_DIM, HID), lambda i, j: (0, 0)),
            pl.BlockSpec((1, OUT_DIM), lambda i, j: (0, 0)),
            pl.BlockSpec((1, OUT_DIM), lambda i, j: (0, 0)),
        ],
        out_specs=pl.BlockSpec((1, BLK_M, OUT_DIM), lambda i, j: (i, j, 0)),
        out_shape=jax.ShapeDtypeStruct((b, m, OUT_DIM), jnp.float32),
        scratch_shapes=[
            pltpu.VMEM((2 * HID, IN_DIM), jnp.bfloat16),
            pltpu.VMEM((OUT_DIM, HID), jnp.bfloat16),
        ],
    )(x, sel[0], sel[1], ln_pre_g.reshape(K, ENC), ln_pre_b.reshape(K, ENC),
      w12, w3, ln_post_g.reshape(1, -1), ln_post_b.reshape(1, -1))
    aux_loss = jnp.zeros((), jnp.float32)
    return (out, aux_loss)


# speedup vs baseline: 1.6562x; 1.6562x over previous
import jax
import jax.numpy as jnp
import numpy as np
from jax.experimental import pallas as pl
from jax.experimental.pallas import tpu as pltpu

K = 2
ENC = 1024
IN_DIM = 2048
OUT_DIM = 4096
HID = 512
BLK_M = 512
SEQ_BLK = K * BLK_M

_SEL = np.zeros((K, BLK_M, SEQ_BLK), dtype=np.float32)
for _r in range(BLK_M):
    _SEL[0, _r, K * _r] = 1.0
    _SEL[1, _r, K * _r + 1] = 1.0


def _fused_kernel(x_ref, se_ref, so_ref, g1_ref, b1_ref, w12_ref, w3_ref,
                  g2_ref, b2_ref, out_ref, w12b_ref):
    @pl.when(jnp.logical_and(pl.program_id(0) == 0, pl.program_id(1) == 0))
    def _cast_weights():
        w12b_ref[...] = w12_ref[...].astype(jnp.bfloat16)

    xb = x_ref[0].astype(jnp.bfloat16)
    xe = jax.lax.dot_general(se_ref[...], xb, (((1,), (0,)), ((), ())),
                             preferred_element_type=jnp.float32)
    xo = jax.lax.dot_general(so_ref[...], xb, (((1,), (0,)), ((), ())),
                             preferred_element_type=jnp.float32)
    mean = (jnp.sum(xe, axis=-1, keepdims=True)
            + jnp.sum(xo, axis=-1, keepdims=True)) * (1.0 / IN_DIM)
    ex2 = (jnp.sum(xe * xe, axis=-1, keepdims=True)
           + jnp.sum(xo * xo, axis=-1, keepdims=True)) * (1.0 / IN_DIM)
    var = ex2 - mean * mean
    rs = jax.lax.rsqrt(var + 1e-6)
    xne = ((xe - mean) * rs * g1_ref[0:1, :] + b1_ref[0:1, :]).astype(
        jnp.bfloat16)
    xno = ((xo - mean) * rs * g1_ref[1:2, :] + b1_ref[1:2, :]).astype(
        jnp.bfloat16)
    h = (jax.lax.dot_general(xne, w12b_ref[:, :ENC], (((1,), (1,)), ((), ())),
                             preferred_element_type=jnp.float32)
         + jax.lax.dot_general(xno, w12b_ref[:, ENC:],
                               (((1,), (1,)), ((), ())),
                               preferred_element_type=jnp.float32))
    gate = h[:, :HID]
    val = h[:, HID:]
    act = (gate * jax.nn.sigmoid(gate) * val).astype(jnp.bfloat16)
    y = jax.lax.dot_general(act, w3_ref[...].astype(jnp.bfloat16),
                            (((1,), (1,)), ((), ())),
                            preferred_element_type=jnp.float32)
    mean2 = jnp.mean(y, axis=-1, keepdims=True)
    yc = y - mean2
    var2 = jnp.mean(yc * yc, axis=-1, keepdims=True)
    out_ref[0] = yc * jax.lax.rsqrt(var2 + 1e-6) * g2_ref[...] + b2_ref[...]


def kernel(x, ln_pre_g, ln_pre_b, w12, w3, router_w, router_b, ln_post_g,
           ln_post_b):
    b, s, d = x.shape
    m = s // K
    nb = m // BLK_M
    sel = jnp.asarray(_SEL, dtype=jnp.bfloat16)
    out = pl.pallas_call(
        _fused_kernel,
        grid=(b, nb),
        in_specs=[
            pl.BlockSpec((1, SEQ_BLK, ENC), lambda i, j: (i, j, 0)),
            pl.BlockSpec((BLK_M, SEQ_BLK), lambda i, j: (0, 0)),
            pl.BlockSpec((BLK_M, SEQ_BLK), lambda i, j: (0, 0)),
            pl.BlockSpec((K, IN_DIM // K), lambda i, j: (0, 0)),
            pl.BlockSpec((K, IN_DIM // K), lambda i, j: (0, 0)),
            pl.BlockSpec((2 * HID, IN_DIM), lambda i, j: (0, 0)),
            pl.BlockSpec((OUT_DIM, HID), lambda i, j: (0, 0)),
            pl.BlockSpec((1, OUT_DIM), lambda i, j: (0, 0)),
            pl.BlockSpec((1, OUT_DIM), lambda i, j: (0, 0)),
        ],
        out_specs=pl.BlockSpec((1, BLK_M, OUT_DIM), lambda i, j: (i, j, 0)),
        out_shape=jax.ShapeDtypeStruct((b, m, OUT_DIM), jnp.float32),
        scratch_shapes=[
            pltpu.VMEM((2 * HID, IN_DIM), jnp.bfloat16),
        ],
    )(x, sel[0], sel[1], ln_pre_g.reshape(K, ENC), ln_pre_b.reshape(K, ENC),
      w12, w3, ln_post_g.reshape(1, -1), ln_post_b.reshape(1, -1))
    aux_loss = jnp.zeros((), jnp.float32)
    return (out, aux_loss)
